# baseline (speedup 1.0000x reference)
import functools

import jax
import jax.numpy as jnp
from jax import lax
from jax.experimental import pallas as pl
from jax.experimental.pallas import tpu as pltpu
from jax.experimental.pallas import tpu_sc as plsc

N = 10000
E = 320000
D = 128
H = 8
DH = 16

NC = 2
NS = 16
NW = NC * NS
EPT = E // NW
C = 32
NPAIR = EPT // (2 * C)
T = EPT - NPAIR * 2 * C
NP = 10240
RPT = NP // NS


def _qkv_body(x_ref, wq_ref, bq_ref, wk_ref, bk_ref, wv_ref, bv_ref,
              q_ref, kv_ref):
    xx = x_ref[...]
    q_ref[...] = jnp.dot(xx, wq_ref[...], preferred_element_type=jnp.float32) + bq_ref[...]
    kv_ref[:, :D] = jnp.dot(xx, wk_ref[...], preferred_element_type=jnp.float32) + bk_ref[...]
    kv_ref[:, D:] = jnp.dot(xx, wv_ref[...], preferred_element_type=jnp.float32) + bv_ref[...]


def _qkv(x, Wq, bq, Wk, bk, Wv, bv):
    BN = 2000
    w_spec = pl.BlockSpec((D, D), lambda i: (0, 0))
    b_spec = pl.BlockSpec((1, D), lambda i: (0, 0))
    return pl.pallas_call(
        _qkv_body,
        grid=(N // BN,),
        in_specs=[pl.BlockSpec((BN, D), lambda i: (i, 0)),
                  w_spec, b_spec, w_spec, b_spec, w_spec, b_spec],
        out_specs=[pl.BlockSpec((BN, D), lambda i: (i, 0)),
                   pl.BlockSpec((BN, 2 * D), lambda i: (i, 0))],
        out_shape=[jax.ShapeDtypeStruct((N, D), jnp.float32),
                   jax.ShapeDtypeStruct((N, 2 * D), jnp.float32)],
    )(x, Wq, bq, Wk, bk, Wv, bv)


def _ep_body(e_ref, w_ref, b_ref, o_ref):
    o_ref[...] = jnp.dot(e_ref[...], w_ref[...], preferred_element_type=jnp.float32) + b_ref[...]


def _ep(e, We, be):
    BE = 8000
    return pl.pallas_call(
        _ep_body,
        grid=(E // BE,),
        in_specs=[pl.BlockSpec((BE, D), lambda i: (i, 0)),
                  pl.BlockSpec((D, D), lambda i: (0, 0)),
                  pl.BlockSpec((1, D), lambda i: (0, 0))],
        out_specs=pl.BlockSpec((BE, D), lambda i: (i, 0)),
        out_shape=jax.ShapeDtypeStruct((E, D), jnp.float32),
    )(e, We, be)


def _edge_body(q_hbm, kv_hbm, ep_hbm, src_hbm, dst_hbm, acc_out, den_out,
               isA, idA, isB, idB, tis, tid, qbA, qbB, kvA, kvB, ebA, ebB,
               mb, pb, acc_sh, den_sh, sem):
    c = lax.axis_index("c")
    s = lax.axis_index("s")
    lanes = lax.broadcasted_iota(jnp.int32, (16,), 0)
    zl = lanes * 0
    zv = jnp.zeros((16,), jnp.float32)

    r0 = s * RPT

    def zrow(i, _):
        for jj in range(H):
            mb[i, pl.ds(DH * jj, DH)] = zv
        pb[i, :] = zv
        return 0

    lax.fori_loop(0, C, zrow, 0)

    def zinit(t, _):
        pltpu.async_copy(mb, acc_sh.at[pl.ds(r0 + t * C, C)], sem).wait()
        pltpu.async_copy(pb, den_sh.at[pl.ds(r0 + t * C, C)], sem).wait()
        return 0

    lax.fori_loop(0, RPT // C, zinit, 0)
    plsc.subcore_barrier()

    e0 = (c * NS + s) * EPT

    def make_edge(qb_, kvb_, eb_):
        def edge_one(i, _):
            l = jnp.zeros((16,), jnp.float32)
            for h in range(H):
                sl = pl.ds(DH * h, DH)
                prod = qb_[i, sl] * kvb_[i, sl] * eb_[i, sl]
                for m in (8, 4, 2, 1):
                    prod = prod + prod.at[lanes ^ m].get(
                        mode="promise_in_bounds", unique_indices=True)
                l = jnp.where(lanes == h, prod, l)
            l = jnp.clip(l * 0.25, -5.0, 5.0)
            p = jnp.where(lanes < H, jnp.exp(l), 0.0)
            pb[i, :] = p
            for h in range(H):
                ph = p.at[zl + h].get(mode="promise_in_bounds")
                mb[i, pl.ds(DH * h, DH)] = ph * kvb_[i, pl.ds(D + DH * h, DH)]
            return 0
        return edge_one

    def scat(idx_, n):
        pltpu.async_copy(mb.at[pl.ds(0, n)], acc_sh.at[idx_], sem, add=True).wait()
        pltpu.async_copy(pb.at[pl.ds(0, n)], den_sh.at[idx_], sem, add=True).wait()

    def pair(gg, _):
        g0 = e0 + gg * (2 * C)
        g1 = g0 + C
        c0 = pltpu.async_copy(src_hbm.at[pl.ds(g0, C)], isA, sem)
        c1 = pltpu.async_copy(dst_hbm.at[pl.ds(g0, C)], idA, sem)
        c2 = pltpu.async_copy(src_hbm.at[pl.ds(g1, C)], isB, sem)
        c3 = pltpu.async_copy(dst_hbm.at[pl.ds(g1, C)], idB, sem)
        c0.wait()
        c1.wait()
        c2.wait()
        c3.wait()
        gqA = pltpu.async_copy(q_hbm.at[idA], qbA, sem)
        gkA = pltpu.async_copy(kv_hbm.at[isA], kvA, sem)
        geA = pltpu.async_copy(ep_hbm.at[pl.ds(g0, C)], ebA, sem)
        gqB = pltpu.async_copy(q_hbm.at[idB], qbB, sem)
        gkB = pltpu.async_copy(kv_hbm.at[isB], kvB, sem)
        geB = pltpu.async_copy(ep_hbm.at[pl.ds(g1, C)], ebB, sem)
        gqA.wait()
        gkA.wait()
        geA.wait()
        lax.fori_loop(0, C, make_edge(qbA, kvA, ebA), 0)
        scat(idA, C)
        gqB.wait()
        gkB.wait()
        geB.wait()
        lax.fori_loop(0, C, make_edge(qbB, kvB, ebB), 0)
        scat(idB, C)
        return 0

    lax.fori_loop(0, NPAIR, pair, 0)

    tb = e0 + NPAIR * 2 * C
    t0 = pltpu.async_copy(src_hbm.at[pl.ds(tb, T)], tis, sem)
    t1 = pltpu.async_copy(dst_hbm.at[pl.ds(tb, T)], tid, sem)
    t0.wait()
    t1.wait()
    tq = pltpu.async_copy(q_hbm.at[tid], qbA.at[pl.ds(0, T)], sem)
    tk = pltpu.async_copy(kv_hbm.at[tis], kvA.at[pl.ds(0, T)], sem)
    te = pltpu.async_copy(ep_hbm.at[pl.ds(tb, T)], ebA.at[pl.ds(0, T)], sem)
    tq.wait()
    tk.wait()
    te.wait()
    lax.fori_loop(0, T, make_edge(qbA, kvA, ebA), 0)
    scat(tid, T)

    plsc.subcore_barrier()

    def wb(t, _):
        rr = r0 + t * C
        pltpu.async_copy(acc_sh.at[pl.ds(rr, C)], mb, sem).wait()
        pltpu.async_copy(mb, acc_out.at[c, pl.ds(rr, C)], sem).wait()
        pltpu.async_copy(den_sh.at[pl.ds(rr, C)], pb, sem).wait()
        pltpu.async_copy(pb, den_out.at[c, pl.ds(rr, C)], sem).wait()
        return 0

    lax.fori_loop(0, RPT // C, wb, 0)


def _edge_pass(q, kv, ep, src, dst):
    mesh = plsc.VectorSubcoreMesh(core_axis_name="c", subcore_axis_name="s",
                                  num_cores=NC, num_subcores=NS)
    fn = pl.kernel(
        _edge_body,
        out_type=[jax.ShapeDtypeStruct((NC, NP, D), jnp.float32),
                  jax.ShapeDtypeStruct((NC, NP, DH), jnp.float32)],
        mesh=mesh,
        compiler_params=pltpu.CompilerParams(use_tc_tiling_on_sc=False),
        scratch_types=[
            pltpu.VMEM((C,), jnp.int32),
            pltpu.VMEM((C,), jnp.int32),
            pltpu.VMEM((C,), jnp.int32),
            pltpu.VMEM((C,), jnp.int32),
            pltpu.VMEM((T,), jnp.int32),
            pltpu.VMEM((T,), jnp.int32),
            pltpu.VMEM((C, D), jnp.float32),
            pltpu.VMEM((C, D), jnp.float32),
            pltpu.VMEM((C, 2 * D), jnp.float32),
            pltpu.VMEM((C, 2 * D), jnp.float32),
            pltpu.VMEM((C, D), jnp.float32),
            pltpu.VMEM((C, D), jnp.float32),
            pltpu.VMEM((C, D), jnp.float32),
            pltpu.VMEM((C, DH), jnp.float32),
            pltpu.VMEM_SHARED((NP, D), jnp.float32),
            pltpu.VMEM_SHARED((NP, DH), jnp.float32),
            pltpu.SemaphoreType.DMA,
        ],
    )
    return fn(q, kv, ep, src, dst)


def _fin_body(acc_ref, den_ref, x_ref, wo_ref, bo_ref, g_ref, b_ref, o_ref):
    den = den_ref[0] + den_ref[1]
    acc = acc_ref[0] + acc_ref[1]
    row = lax.broadcasted_iota(jnp.int32, (DH, D), 0)
    col = lax.broadcasted_iota(jnp.int32, (DH, D), 1)
    erep = (col // DH == row).astype(jnp.float32)
    den128 = jnp.dot(den, erep, preferred_element_type=jnp.float32)
    attn = acc / (den128 + 1e-16)
    out = (jnp.dot(attn, wo_ref[...], preferred_element_type=jnp.float32)
           + bo_ref[...] + x_ref[...])
    mu = jnp.mean(out, axis=1, keepdims=True)
    dlt = out - mu
    var = jnp.mean(dlt * dlt, axis=1, keepdims=True)
    o_ref[...] = dlt * lax.rsqrt(var + 1e-5) * g_ref[...] + b_ref[...]


def _final(acc2, den2, x, Wo, bo, ln_g, ln_b):
    BN = 2000
    return pl.pallas_call(
        _fin_body,
        grid=(N // BN,),
        in_specs=[pl.BlockSpec((NC, BN, D), lambda i: (0, i, 0)),
                  pl.BlockSpec((NC, BN, DH), lambda i: (0, i, 0)),
                  pl.BlockSpec((BN, D), lambda i: (i, 0)),
                  pl.BlockSpec((D, D), lambda i: (0, 0)),
                  pl.BlockSpec((1, D), lambda i: (0, 0)),
                  pl.BlockSpec((1, D), lambda i: (0, 0)),
                  pl.BlockSpec((1, D), lambda i: (0, 0))],
        out_specs=pl.BlockSpec((BN, D), lambda i: (i, 0)),
        out_shape=jax.ShapeDtypeStruct((N, D), jnp.float32),
    )(acc2, den2, x, Wo, bo, ln_g, ln_b)


def kernel(x, e, edge_index, Wq, bq, Wk, bk, Wv, bv, We, be, Wo, bo, ln_g, ln_b):
    ei = edge_index.astype(jnp.int32)
    src = ei[0]
    dst = ei[1]
    q, kv = _qkv(x, Wq, bq.reshape(1, D), Wk, bk.reshape(1, D),
                 Wv, bv.reshape(1, D))
    ep = _ep(e, We, be.reshape(1, D))
    acc2, den2 = _edge_pass(q, kv, ep, src, dst)
    return _final(acc2, den2, x, Wo, bo.reshape(1, D),
                  ln_g.reshape(1, D), ln_b.reshape(1, D))


# speedup vs baseline: 1.0538x; 1.0538x over previous
import functools

import jax
import jax.numpy as jnp
from jax import lax
from jax.experimental import pallas as pl
from jax.experimental.pallas import tpu as pltpu
from jax.experimental.pallas import tpu_sc as plsc

N = 10000
E = 320000
D = 128
H = 8
DH = 16

NC = 2
NS = 16
NW = NC * NS
EPT = E // NW
C = 40
NCHUNK = EPT // C
NP = 10240
RPT = NP // NS


def _qkv_body(x_ref, wq_ref, bq_ref, wk_ref, bk_ref, wv_ref, bv_ref,
              q_ref, k_ref, v_ref):
    xx = x_ref[...]
    q_ref[...] = jnp.dot(xx, wq_ref[...], preferred_element_type=jnp.float32) + bq_ref[...]
    k_ref[...] = jnp.dot(xx, wk_ref[...], preferred_element_type=jnp.float32) + bk_ref[...]
    v_ref[...] = jnp.dot(xx, wv_ref[...], preferred_element_type=jnp.float32) + bv_ref[...]


def _qkv(x, Wq, bq, Wk, bk, Wv, bv):
    BN = 2000
    w_spec = pl.BlockSpec((D, D), lambda i: (0, 0))
    b_spec = pl.BlockSpec((1, D), lambda i: (0, 0))
    r_spec = pl.BlockSpec((BN, D), lambda i: (i, 0))
    return pl.pallas_call(
        _qkv_body,
        grid=(N // BN,),
        in_specs=[r_spec, w_spec, b_spec, w_spec, b_spec, w_spec, b_spec],
        out_specs=[r_spec, r_spec, r_spec],
        out_shape=[jax.ShapeDtypeStruct((N, D), jnp.float32)] * 3,
    )(x, Wq, bq, Wk, bk, Wv, bv)


def _ep_body(e_ref, w_ref, b_ref, o_ref):
    o_ref[...] = jnp.dot(e_ref[...], w_ref[...], preferred_element_type=jnp.float32) + b_ref[...]


def _ep(e, We, be):
    BE = 8000
    return pl.pallas_call(
        _ep_body,
        grid=(E // BE,),
        in_specs=[pl.BlockSpec((BE, D), lambda i: (i, 0)),
                  pl.BlockSpec((D, D), lambda i: (0, 0)),
                  pl.BlockSpec((1, D), lambda i: (0, 0))],
        out_specs=pl.BlockSpec((BE, D), lambda i: (i, 0)),
        out_shape=jax.ShapeDtypeStruct((E, D), jnp.float32),
    )(e, We, be)


def _edge_body(q_hbm, k_hbm, v_hbm, ep_hbm, src_hbm, dst_hbm, acc_out, den_out,
               idx_s, idx_d, qb, kb, vb, eb, pb, acc_sh, den_sh, sem):
    c = lax.axis_index("c")
    s = lax.axis_index("s")
    lanes = lax.broadcasted_iota(jnp.int32, (16,), 0)
    zl = lanes * 0
    zv = jnp.zeros((16,), jnp.float32)

    r0 = s * RPT

    def zrow(i, _):
        for jj in range(H):
            qb[i, pl.ds(DH * jj, DH)] = zv
        pb[i, :] = zv
        return 0

    lax.fori_loop(0, C, zrow, 0)

    def zinit(t, _):
        pltpu.async_copy(qb, acc_sh.at[pl.ds(r0 + t * C, C)], sem).wait()
        pltpu.async_copy(pb, den_sh.at[pl.ds(r0 + t * C, C)], sem).wait()
        return 0

    lax.fori_loop(0, RPT // C, zinit, 0)
    plsc.subcore_barrier()

    e0 = (c * NS + s) * EPT

    def edge_one(i, _):
        l = jnp.zeros((16,), jnp.float32)
        for h in range(H):
            sl = pl.ds(DH * h, DH)
            prod = qb[i, sl] * kb[i, sl] * eb[i, sl]
            for m in (8, 4, 2, 1):
                prod = prod + prod.at[lanes ^ m].get(mode="promise_in_bounds", unique_indices=True)
            l = jnp.where(lanes == h, prod, l)
        l = jnp.clip(l * 0.25, -5.0, 5.0)
        p = jnp.where(lanes < H, jnp.exp(l), 0.0)
        pb[i, :] = p
        for h in range(H):
            sl = pl.ds(DH * h, DH)
            ph = p.at[zl + h].get(mode="promise_in_bounds")
            kb[i, sl] = ph * vb[i, sl]
        return 0

    def chunk(g, _):
        base = e0 + g * C
        ci = pltpu.async_copy(src_hbm.at[pl.ds(base, C)], idx_s, sem)
        cj = pltpu.async_copy(dst_hbm.at[pl.ds(base, C)], idx_d, sem)
        ci.wait()
        cj.wait()
        cq = pltpu.async_copy(q_hbm.at[idx_d], qb, sem)
        ck = pltpu.async_copy(k_hbm.at[idx_s], kb, sem)
        cv = pltpu.async_copy(v_hbm.at[idx_s], vb, sem)
        ce = pltpu.async_copy(ep_hbm.at[pl.ds(base, C)], eb, sem)
        cq.wait()
        ck.wait()
        cv.wait()
        ce.wait()
        lax.fori_loop(0, C, edge_one, 0)
        pltpu.async_copy(kb, acc_sh.at[idx_d], sem, add=True).wait()
        pltpu.async_copy(pb, den_sh.at[idx_d], sem, add=True).wait()
        return 0

    lax.fori_loop(0, NCHUNK, chunk, 0)

    plsc.subcore_barrier()

    def wb(t, _):
        rr = r0 + t * C
        pltpu.async_copy(acc_sh.at[pl.ds(rr, C)], qb, sem).wait()
        pltpu.async_copy(qb, acc_out.at[c, pl.ds(rr, C)], sem).wait()
        pltpu.async_copy(den_sh.at[pl.ds(rr, C)], pb, sem).wait()
        pltpu.async_copy(pb, den_out.at[c, pl.ds(rr, C)], sem).wait()
        return 0

    lax.fori_loop(0, RPT // C, wb, 0)


def _edge_pass(q, k, v, ep, src, dst):
    mesh = plsc.VectorSubcoreMesh(core_axis_name="c", subcore_axis_name="s",
                                  num_cores=NC, num_subcores=NS)
    fn = pl.kernel(
        _edge_body,
        out_type=[jax.ShapeDtypeStruct((NC, NP, D), jnp.float32),
                  jax.ShapeDtypeStruct((NC, NP, DH), jnp.float32)],
        mesh=mesh,
        compiler_params=pltpu.CompilerParams(use_tc_tiling_on_sc=False),
        scratch_types=[
            pltpu.VMEM((C,), jnp.int32),
            pltpu.VMEM((C,), jnp.int32),
            pltpu.VMEM((C, D), jnp.float32),
            pltpu.VMEM((C, D), jnp.float32),
            pltpu.VMEM((C, D), jnp.float32),
            pltpu.VMEM((C, D), jnp.float32),
            pltpu.VMEM((C, DH), jnp.float32),
            pltpu.VMEM_SHARED((NP, D), jnp.float32),
            pltpu.VMEM_SHARED((NP, DH), jnp.float32),
            pltpu.SemaphoreType.DMA,
        ],
    )
    return fn(q, k, v, ep, src, dst)


def _fin_body(acc_ref, den_ref, x_ref, wo_ref, bo_ref, g_ref, b_ref, o_ref):
    den = den_ref[0] + den_ref[1]
    acc = acc_ref[0] + acc_ref[1]
    row = lax.broadcasted_iota(jnp.int32, (DH, D), 0)
    col = lax.broadcasted_iota(jnp.int32, (DH, D), 1)
    erep = (col // DH == row).astype(jnp.float32)
    den128 = jnp.dot(den, erep, preferred_element_type=jnp.float32)
    attn = acc / (den128 + 1e-16)
    out = (jnp.dot(attn, wo_ref[...], preferred_element_type=jnp.float32)
           + bo_ref[...] + x_ref[...])
    mu = jnp.mean(out, axis=1, keepdims=True)
    dlt = out - mu
    var = jnp.mean(dlt * dlt, axis=1, keepdims=True)
    o_ref[...] = dlt * lax.rsqrt(var + 1e-5) * g_ref[...] + b_ref[...]


def _final(acc2, den2, x, Wo, bo, ln_g, ln_b):
    BN = 2000
    return pl.pallas_call(
        _fin_body,
        grid=(N // BN,),
        in_specs=[pl.BlockSpec((NC, BN, D), lambda i: (0, i, 0)),
                  pl.BlockSpec((NC, BN, DH), lambda i: (0, i, 0)),
                  pl.BlockSpec((BN, D), lambda i: (i, 0)),
                  pl.BlockSpec((D, D), lambda i: (0, 0)),
                  pl.BlockSpec((1, D), lambda i: (0, 0)),
                  pl.BlockSpec((1, D), lambda i: (0, 0)),
                  pl.BlockSpec((1, D), lambda i: (0, 0))],
        out_specs=pl.BlockSpec((BN, D), lambda i: (i, 0)),
        out_shape=jax.ShapeDtypeStruct((N, D), jnp.float32),
    )(acc2, den2, x, Wo, bo, ln_g, ln_b)


def kernel(x, e, edge_index, Wq, bq, Wk, bk, Wv, bv, We, be, Wo, bo, ln_g, ln_b):
    ei = edge_index.astype(jnp.int32)
    src = ei[0]
    dst = ei[1]
    q, k, v = _qkv(x, Wq, bq.reshape(1, D), Wk, bk.reshape(1, D),
                   Wv, bv.reshape(1, D))
    ep = _ep(e, We, be.reshape(1, D))
    acc2, den2 = _edge_pass(q, k, v, ep, src, dst)
    return _final(acc2, den2, x, Wo, bo.reshape(1, D),
                  ln_g.reshape(1, D), ln_b.reshape(1, D))


# speedup vs baseline: 1.0683x; 1.0137x over previous
import functools

import jax
import jax.numpy as jnp
from jax import lax
from jax.experimental import pallas as pl
from jax.experimental.pallas import tpu as pltpu
from jax.experimental.pallas import tpu_sc as plsc

N = 10000
E = 320000
D = 128
H = 8
DH = 16

NC = 2
NS = 16
NW = NC * NS
EPT = E // NW
C = 40
NCHUNK = EPT // C
NP = 10240
RPT = NP // NS


def _qkv_body(x_ref, wq_ref, bq_ref, wk_ref, bk_ref, wv_ref, bv_ref,
              q_ref, k_ref, v_ref):
    xx = x_ref[...]
    q_ref[...] = jnp.dot(xx, wq_ref[...], preferred_element_type=jnp.float32) + bq_ref[...]
    k_ref[...] = jnp.dot(xx, wk_ref[...], preferred_element_type=jnp.float32) + bk_ref[...]
    v_ref[...] = jnp.dot(xx, wv_ref[...], preferred_element_type=jnp.float32) + bv_ref[...]


def _qkv(x, Wq, bq, Wk, bk, Wv, bv):
    BN = 2000
    w_spec = pl.BlockSpec((D, D), lambda i: (0, 0))
    b_spec = pl.BlockSpec((1, D), lambda i: (0, 0))
    r_spec = pl.BlockSpec((BN, D), lambda i: (i, 0))
    return pl.pallas_call(
        _qkv_body,
        grid=(N // BN,),
        in_specs=[r_spec, w_spec, b_spec, w_spec, b_spec, w_spec, b_spec],
        out_specs=[r_spec, r_spec, r_spec],
        out_shape=[jax.ShapeDtypeStruct((N, D), jnp.float32)] * 3,
    )(x, Wq, bq, Wk, bk, Wv, bv)


def _ep_body(e_ref, w_ref, b_ref, o_ref):
    o_ref[...] = jnp.dot(e_ref[...], w_ref[...], preferred_element_type=jnp.float32) + b_ref[...]


def _ep(e, We, be):
    BE = 8000
    return pl.pallas_call(
        _ep_body,
        grid=(E // BE,),
        in_specs=[pl.BlockSpec((BE, D), lambda i: (i, 0)),
                  pl.BlockSpec((D, D), lambda i: (0, 0)),
                  pl.BlockSpec((1, D), lambda i: (0, 0))],
        out_specs=pl.BlockSpec((BE, D), lambda i: (i, 0)),
        out_shape=jax.ShapeDtypeStruct((E, D), jnp.float32),
    )(e, We, be)


def _edge_body(q_hbm, k_hbm, v_hbm, ep_hbm, src_hbm, dst_hbm, acc_out, den_out,
               idx_s, idx_d, qb, kb, vb, eb, pb, acc_sh, den_sh, sem):
    c = lax.axis_index("c")
    s = lax.axis_index("s")
    lanes = lax.broadcasted_iota(jnp.int32, (16,), 0)
    zl = lanes * 0
    zv = jnp.zeros((16,), jnp.float32)

    r0 = s * RPT

    def zrow(i, _):
        for jj in range(H):
            qb[i, pl.ds(DH * jj, DH)] = zv
        pb[i, :] = zv
        return 0

    lax.fori_loop(0, C, zrow, 0)

    def zinit(t, _):
        pltpu.async_copy(qb, acc_sh.at[pl.ds(r0 + t * C, C)], sem).wait()
        pltpu.async_copy(pb, den_sh.at[pl.ds(r0 + t * C, C)], sem).wait()
        return 0

    lax.fori_loop(0, RPT // C, zinit, 0)
    plsc.subcore_barrier()

    e0 = (c * NS + s) * EPT

    def edge_one(i, _):
        l = jnp.zeros((16,), jnp.float32)
        for h in range(H):
            sl = pl.ds(DH * h, DH)
            prod = qb[i, sl] * kb[i, sl] * eb[i, sl]
            for m in (8, 4, 2, 1):
                prod = prod + prod.at[lanes ^ m].get(mode="promise_in_bounds", unique_indices=True)
            l = jnp.where(lanes == h, prod, l)
        l = jnp.clip(l * 0.25, -5.0, 5.0)
        p = jnp.where(lanes < H, jnp.exp(l), 0.0)
        pb[i, :] = p
        for h in range(H):
            sl = pl.ds(DH * h, DH)
            ph = p.at[zl + h].get(mode="promise_in_bounds")
            kb[i, sl] = ph * vb[i, sl]
        return 0

    def chunk(g, _):
        base = e0 + g * C
        ci = pltpu.async_copy(src_hbm.at[pl.ds(base, C)], idx_s, sem)
        cj = pltpu.async_copy(dst_hbm.at[pl.ds(base, C)], idx_d, sem)
        ci.wait()
        cj.wait()
        cq = pltpu.async_copy(q_hbm.at[idx_d], qb, sem)
        ck = pltpu.async_copy(k_hbm.at[idx_s], kb, sem)
        cv = pltpu.async_copy(v_hbm.at[idx_s], vb, sem)
        ce = pltpu.async_copy(ep_hbm.at[pl.ds(base, C)], eb, sem)
        cq.wait()
        ck.wait()
        cv.wait()
        ce.wait()
        lax.fori_loop(0, C, edge_one, 0)
        s1 = pltpu.async_copy(kb, acc_sh.at[idx_d], sem, add=True)
        s2 = pltpu.async_copy(pb, den_sh.at[idx_d], sem, add=True)
        s1.wait()
        s2.wait()
        return 0

    lax.fori_loop(0, NCHUNK, chunk, 0)

    plsc.subcore_barrier()

    def wb(t, _):
        rr = r0 + t * C
        pltpu.async_copy(acc_sh.at[pl.ds(rr, C)], qb, sem).wait()
        pltpu.async_copy(qb, acc_out.at[c, pl.ds(rr, C)], sem).wait()
        pltpu.async_copy(den_sh.at[pl.ds(rr, C)], pb, sem).wait()
        pltpu.async_copy(pb, den_out.at[c, pl.ds(rr, C)], sem).wait()
        return 0

    lax.fori_loop(0, RPT // C, wb, 0)


def _edge_pass(q, k, v, ep, src, dst):
    mesh = plsc.VectorSubcoreMesh(core_axis_name="c", subcore_axis_name="s",
                                  num_cores=NC, num_subcores=NS)
    fn = pl.kernel(
        _edge_body,
        out_type=[jax.ShapeDtypeStruct((NC, NP, D), jnp.float32),
                  jax.ShapeDtypeStruct((NC, NP, DH), jnp.float32)],
        mesh=mesh,
        compiler_params=pltpu.CompilerParams(use_tc_tiling_on_sc=False),
        scratch_types=[
            pltpu.VMEM((C,), jnp.int32),
            pltpu.VMEM((C,), jnp.int32),
            pltpu.VMEM((C, D), jnp.float32),
            pltpu.VMEM((C, D), jnp.float32),
            pltpu.VMEM((C, D), jnp.float32),
            pltpu.VMEM((C, D), jnp.float32),
            pltpu.VMEM((C, DH), jnp.float32),
            pltpu.VMEM_SHARED((NP, D), jnp.float32),
            pltpu.VMEM_SHARED((NP, DH), jnp.float32),
            pltpu.SemaphoreType.DMA,
        ],
    )
    return fn(q, k, v, ep, src, dst)


def _fin_body(acc_ref, den_ref, x_ref, wo_ref, bo_ref, g_ref, b_ref, o_ref):
    den = den_ref[0] + den_ref[1]
    acc = acc_ref[0] + acc_ref[1]
    row = lax.broadcasted_iota(jnp.int32, (DH, D), 0)
    col = lax.broadcasted_iota(jnp.int32, (DH, D), 1)
    erep = (col // DH == row).astype(jnp.float32)
    den128 = jnp.dot(den, erep, preferred_element_type=jnp.float32)
    attn = acc / (den128 + 1e-16)
    out = (jnp.dot(attn, wo_ref[...], preferred_element_type=jnp.float32)
           + bo_ref[...] + x_ref[...])
    mu = jnp.mean(out, axis=1, keepdims=True)
    dlt = out - mu
    var = jnp.mean(dlt * dlt, axis=1, keepdims=True)
    o_ref[...] = dlt * lax.rsqrt(var + 1e-5) * g_ref[...] + b_ref[...]


def _final(acc2, den2, x, Wo, bo, ln_g, ln_b):
    BN = 2000
    return pl.pallas_call(
        _fin_body,
        grid=(N // BN,),
        in_specs=[pl.BlockSpec((NC, BN, D), lambda i: (0, i, 0)),
                  pl.BlockSpec((NC, BN, DH), lambda i: (0, i, 0)),
                  pl.BlockSpec((BN, D), lambda i: (i, 0)),
                  pl.BlockSpec((D, D), lambda i: (0, 0)),
                  pl.BlockSpec((1, D), lambda i: (0, 0)),
                  pl.BlockSpec((1, D), lambda i: (0, 0)),
                  pl.BlockSpec((1, D), lambda i: (0, 0))],
        out_specs=pl.BlockSpec((BN, D), lambda i: (i, 0)),
        out_shape=jax.ShapeDtypeStruct((N, D), jnp.float32),
    )(acc2, den2, x, Wo, bo, ln_g, ln_b)


def kernel(x, e, edge_index, Wq, bq, Wk, bk, Wv, bv, We, be, Wo, bo, ln_g, ln_b):
    ei = edge_index.astype(jnp.int32)
    src = ei[0]
    dst = ei[1]
    q, k, v = _qkv(x, Wq, bq.reshape(1, D), Wk, bk.reshape(1, D),
                   Wv, bv.reshape(1, D))
    ep = _ep(e, We, be.reshape(1, D))
    acc2, den2 = _edge_pass(q, k, v, ep, src, dst)
    return _final(acc2, den2, x, Wo, bo.reshape(1, D),
                  ln_g.reshape(1, D), ln_b.reshape(1, D))


# speedup vs baseline: 1.0695x; 1.0011x over previous
import jax
import jax.numpy as jnp
from jax import lax
from jax.experimental import pallas as pl
from jax.experimental.pallas import tpu as pltpu
from jax.experimental.pallas import tpu_sc as plsc

N = 10000
E = 320000
D = 128
H = 8
DH = 16

NC = 2
NS = 16
NW = NC * NS
EPT = E // NW
C = 40
NCHUNK = EPT // C
NP = 10240
RPT = NP // NS


def _qkv_body(x_ref, wq_ref, bq_ref, wk_ref, bk_ref, wv_ref, bv_ref,
              q_ref, k_ref, v_ref):
    xx = x_ref[...]
    q_ref[...] = jnp.dot(xx, wq_ref[...], preferred_element_type=jnp.float32) + bq_ref[...]
    k_ref[...] = jnp.dot(xx, wk_ref[...], preferred_element_type=jnp.float32) + bk_ref[...]
    v_ref[...] = jnp.dot(xx, wv_ref[...], preferred_element_type=jnp.float32) + bv_ref[...]


def _qkv(x, Wq, bq, Wk, bk, Wv, bv):
    BN = 2000
    w_spec = pl.BlockSpec((D, D), lambda i: (0, 0))
    b_spec = pl.BlockSpec((1, D), lambda i: (0, 0))
    r_spec = pl.BlockSpec((BN, D), lambda i: (i, 0))
    return pl.pallas_call(
        _qkv_body,
        grid=(N // BN,),
        in_specs=[r_spec, w_spec, b_spec, w_spec, b_spec, w_spec, b_spec],
        out_specs=[r_spec, r_spec, r_spec],
        out_shape=[jax.ShapeDtypeStruct((N, D), jnp.float32)] * 3,
    )(x, Wq, bq, Wk, bk, Wv, bv)


def _ep_body(e_ref, w_ref, b_ref, o_ref):
    o_ref[...] = jnp.dot(e_ref[...], w_ref[...], preferred_element_type=jnp.float32) + b_ref[...]


def _ep(e, We, be):
    BE = 8000
    return pl.pallas_call(
        _ep_body,
        grid=(E // BE,),
        in_specs=[pl.BlockSpec((BE, D), lambda i: (i, 0)),
                  pl.BlockSpec((D, D), lambda i: (0, 0)),
                  pl.BlockSpec((1, D), lambda i: (0, 0))],
        out_specs=pl.BlockSpec((BE, D), lambda i: (i, 0)),
        out_shape=jax.ShapeDtypeStruct((E, D), jnp.float32),
    )(e, We, be)


def _edge_body(q_hbm, k_hbm, v_hbm, ep_hbm, src_hbm, dst_hbm, acc_out, den_out,
               idx_s, idx_d, qb, kb, vb, eb, pb, acc_sh, den_sh, sem):
    c = lax.axis_index("c")
    s = lax.axis_index("s")
    lanes = lax.broadcasted_iota(jnp.int32, (16,), 0)
    zl = lanes * 0
    zv = jnp.zeros((16,), jnp.float32)

    r0 = s * RPT

    def zrow(i, _):
        for jj in range(H):
            qb[i, pl.ds(DH * jj, DH)] = zv
        pb[i, :] = zv
        return 0

    lax.fori_loop(0, C, zrow, 0)

    def zinit(t, _):
        pltpu.async_copy(qb, acc_sh.at[pl.ds(r0 + t * C, C)], sem).wait()
        pltpu.async_copy(pb, den_sh.at[pl.ds(r0 + t * C, C)], sem).wait()
        return 0

    lax.fori_loop(0, RPT // C, zinit, 0)
    plsc.subcore_barrier()

    e0 = (c * NS + s) * EPT

    def edge_one(i, _):
        l = jnp.zeros((16,), jnp.float32)
        for h in range(H):
            sl = pl.ds(DH * h, DH)
            prod = qb[i, sl] * kb[i, sl] * eb[i, sl]
            for m in (8, 4, 2, 1):
                prod = prod + prod.at[lanes ^ m].get(mode="promise_in_bounds", unique_indices=True)
            l = jnp.where(lanes == h, prod, l)
        l = jnp.clip(l * 0.25, -5.0, 5.0)
        p = jnp.where(lanes < H, jnp.exp(l), 0.0)
        pb[i, :] = p
        for h in range(H):
            sl = pl.ds(DH * h, DH)
            ph = p.at[zl + h].get(mode="promise_in_bounds")
            kb[i, sl] = ph * vb[i, sl]
        return 0

    def chunk(g, _):
        base = e0 + g * C
        ci = pltpu.async_copy(src_hbm.at[pl.ds(base, C)], idx_s, sem)
        cj = pltpu.async_copy(dst_hbm.at[pl.ds(base, C)], idx_d, sem)
        ci.wait()
        cj.wait()
        cq = pltpu.async_copy(q_hbm.at[idx_d], qb, sem)
        ck = pltpu.async_copy(k_hbm.at[idx_s], kb, sem)
        cv = pltpu.async_copy(v_hbm.at[idx_s], vb, sem)
        ce = pltpu.async_copy(ep_hbm.at[pl.ds(base, C)], eb, sem)
        cq.wait()
        ck.wait()
        cv.wait()
        ce.wait()
        lax.fori_loop(0, C, edge_one, 0)
        s1 = pltpu.async_copy(kb, acc_sh.at[idx_d], sem, add=True)
        s2 = pltpu.async_copy(pb, den_sh.at[idx_d], sem, add=True)
        s1.wait()
        s2.wait()
        return 0

    lax.fori_loop(0, NCHUNK, chunk, 0)

    plsc.subcore_barrier()

    def wb(t, _):
        rr = r0 + t * C
        pltpu.async_copy(acc_sh.at[pl.ds(rr, C)], qb, sem).wait()
        pltpu.async_copy(qb, acc_out.at[c, pl.ds(rr, C)], sem).wait()
        pltpu.async_copy(den_sh.at[pl.ds(rr, C)], pb, sem).wait()
        pltpu.async_copy(pb, den_out.at[c, pl.ds(rr, C)], sem).wait()
        return 0

    lax.fori_loop(0, RPT // C, wb, 0)


def _edge_pass(q, k, v, ep, src, dst):
    mesh = plsc.VectorSubcoreMesh(core_axis_name="c", subcore_axis_name="s",
                                  num_cores=NC, num_subcores=NS)
    fn = pl.kernel(
        _edge_body,
        out_type=[jax.ShapeDtypeStruct((NC, NP, D), jnp.float32),
                  jax.ShapeDtypeStruct((NC, NP, DH), jnp.float32)],
        mesh=mesh,
        compiler_params=pltpu.CompilerParams(use_tc_tiling_on_sc=False),
        scratch_types=[
            pltpu.VMEM((C,), jnp.int32),
            pltpu.VMEM((C,), jnp.int32),
            pltpu.VMEM((C, D), jnp.float32),
            pltpu.VMEM((C, D), jnp.float32),
            pltpu.VMEM((C, D), jnp.float32),
            pltpu.VMEM((C, D), jnp.float32),
            pltpu.VMEM((C, DH), jnp.float32),
            pltpu.VMEM_SHARED((NP, D), jnp.float32),
            pltpu.VMEM_SHARED((NP, DH), jnp.float32),
            pltpu.SemaphoreType.DMA,
        ],
    )
    return fn(q, k, v, ep, src, dst)


def _fin_body(acc_ref, den_ref, x_ref, wo_ref, bo_ref, g_ref, b_ref, o_ref):
    den = den_ref[0] + den_ref[1]
    acc = acc_ref[0] + acc_ref[1]
    row = lax.broadcasted_iota(jnp.int32, (DH, D), 0)
    col = lax.broadcasted_iota(jnp.int32, (DH, D), 1)
    erep = (col // DH == row).astype(jnp.float32)
    den128 = jnp.dot(den, erep, preferred_element_type=jnp.float32)
    attn = acc / (den128 + 1e-16)
    out = (jnp.dot(attn, wo_ref[...], preferred_element_type=jnp.float32)
           + bo_ref[...] + x_ref[...])
    mu = jnp.mean(out, axis=1, keepdims=True)
    dlt = out - mu
    var = jnp.mean(dlt * dlt, axis=1, keepdims=True)
    o_ref[...] = dlt * lax.rsqrt(var + 1e-5) * g_ref[...] + b_ref[...]


def _final(acc2, den2, x, Wo, bo, ln_g, ln_b):
    BN = 2000
    return pl.pallas_call(
        _fin_body,
        grid=(N // BN,),
        in_specs=[pl.BlockSpec((NC, BN, D), lambda i: (0, i, 0)),
                  pl.BlockSpec((NC, BN, DH), lambda i: (0, i, 0)),
                  pl.BlockSpec((BN, D), lambda i: (i, 0)),
                  pl.BlockSpec((D, D), lambda i: (0, 0)),
                  pl.BlockSpec((1, D), lambda i: (0, 0)),
                  pl.BlockSpec((1, D), lambda i: (0, 0)),
                  pl.BlockSpec((1, D), lambda i: (0, 0))],
        out_specs=pl.BlockSpec((BN, D), lambda i: (i, 0)),
        out_shape=jax.ShapeDtypeStruct((N, D), jnp.float32),
    )(acc2, den2, x, Wo, bo, ln_g, ln_b)


def kernel(x, e, edge_index, Wq, bq, Wk, bk, Wv, bv, We, be, Wo, bo, ln_g, ln_b):
    ei = edge_index.astype(jnp.int32)
    src = ei[0]
    dst = ei[1]
    q, k, v = _qkv(x, Wq, bq.reshape(1, D), Wk, bk.reshape(1, D),
                   Wv, bv.reshape(1, D))
    ep = _ep(e, We, be.reshape(1, D))
    acc2, den2 = _edge_pass(q, k, v, ep, src, dst)
    return _final(acc2, den2, x, Wo, bo.reshape(1, D),
                  ln_g.reshape(1, D), ln_b.reshape(1, D))
